# baseline (speedup 1.0000x reference)
import jax
import jax.numpy as jnp
from jax.experimental import pallas as pl
from jax.experimental.pallas import tpu as pltpu

B = 8
T = 512
U = 31
U1 = U + 1
V = 512
TBF = 128
NBLK = (B * T) // TBF
D = T + U1 - 1
DP = D + 1
NEG = -1.0e9


def _roll0(x, k):
    return jnp.concatenate([x[x.shape[0] - k:], x[: x.shape[0] - k]], axis=0)


def _fused_kernel(tgt_ref, x_ref, out_ref, oh_s, blank_s, y_s, bsh, ysh):
    pid = pl.program_id(0)
    nb_per_b = T // TBF
    b = pid // nb_per_b
    t0 = (pid % nb_per_b) * TBF

    @pl.when(pid % nb_per_b == 0)
    def _():
        tgt = tgt_ref[pl.ds(b, 1), :]
        iov = jax.lax.broadcasted_iota(jnp.int32, (V, U1), 0)
        oh_s[...] = jnp.where(iov == tgt, 1.0, 0.0).astype(jnp.bfloat16)

    x = x_ref[...]
    m = jnp.max(x, axis=2, keepdims=True)
    s = jnp.sum(jnp.exp(x - m), axis=2, keepdims=True)
    lse = (m + jnp.log(s))[:, :, 0]
    blank = x[:, :, 0] - lse
    xb = x.reshape(TBF * U1, V).astype(jnp.bfloat16)
    r = jax.lax.dot_general(xb, oh_s[...], (((1,), (0,)), ((), ())),
                            preferred_element_type=jnp.float32)
    iu = jax.lax.broadcasted_iota(jnp.int32, (TBF * U1, U1), 1)
    ru = jax.lax.broadcasted_iota(jnp.int32, (TBF * U1, U1), 0) & (U1 - 1)
    yd = jnp.sum(jnp.where(iu == ru, r, 0.0), axis=1).reshape(TBF, U1)
    yv = yd - lse

    blank_s[pl.ds(t0, TBF), pl.ds(b, 1), :] = blank[:, None, :]
    y_s[pl.ds(t0, TBF), pl.ds(b, 1), :] = yv[:, None, :]

    @pl.when(pid == NBLK - 1)
    def _():
        pad = jnp.full((DP - T, B, U1), NEG, jnp.float32)
        bp = jnp.concatenate([blank_s[...], pad], axis=0)
        yp = jnp.concatenate([y_s[...], pad], axis=0)
        iota_u = jax.lax.broadcasted_iota(jnp.int32, (DP, B, U1), 2)
        yp = jnp.where(iota_u == U, NEG, yp)
        for k in (1, 2, 4, 8, 16):
            mask = (iota_u & k) != 0
            bp = jnp.where(mask, _roll0(bp, k), bp)
            yp = jnp.where(mask, _roll0(yp, k), yp)
        bsh[...] = bp
        ysh[...] = yp

        iu3 = jax.lax.broadcasted_iota(jnp.int32, (1, B, U1), 2)
        a0 = jnp.where(iu3 == 0, 0.0, NEG)

        def body(d, a):
            bcol = bsh[pl.ds(d - 1, 1)]
            ycol = ysh[pl.ds(d - 1, 1)]
            c = a + ycol
            cs = jnp.concatenate(
                [jnp.full((1, B, 1), NEG, jnp.float32), c[:, :, :U]], axis=2)
            t1 = a + bcol
            mx = jnp.maximum(t1, cs)
            return mx + jnp.log1p(jnp.exp(-jnp.abs(t1 - cs)))

        a = jax.lax.fori_loop(1, D, body, a0)
        loglik = a[:, :, U1 - 1] + bsh[pl.ds(D - 1, 1)][:, :, U1 - 1]
        out_ref[...] = -jnp.sum(loglik, axis=1, keepdims=True) / B


def kernel(logits, targets, logit_lengths, target_lengths):
    x = logits.reshape(B * T, U1, V)
    tgt = jnp.concatenate(
        [targets.astype(jnp.int32), jnp.full((B, 1), -1, jnp.int32)], axis=1)
    out = pl.pallas_call(
        _fused_kernel,
        grid=(NBLK,),
        in_specs=[
            pl.BlockSpec((B, U1), lambda i: (0, 0)),
            pl.BlockSpec((TBF, U1, V), lambda i: (i, 0, 0)),
        ],
        out_specs=pl.BlockSpec((1, 1), lambda i: (0, 0)),
        out_shape=jax.ShapeDtypeStruct((1, 1), jnp.float32),
        scratch_shapes=[
            pltpu.VMEM((V, U1), jnp.bfloat16),
            pltpu.VMEM((T, B, U1), jnp.float32),
            pltpu.VMEM((T, B, U1), jnp.float32),
            pltpu.VMEM((DP, B, U1), jnp.float32),
            pltpu.VMEM((DP, B, U1), jnp.float32),
        ],
    )(tgt, x)
    return out[0, 0]


# speedup vs baseline: 1.0746x; 1.0746x over previous
import jax
import jax.numpy as jnp
from jax.experimental import pallas as pl
from jax.experimental.pallas import tpu as pltpu

B = 8
T = 512
U = 31
U1 = U + 1
V = 512
TBF = 256
NBLK = (B * T) // TBF
D = T + U1 - 1
DP = D + 1
NEG = -1.0e9


def _roll0(x, k):
    return jnp.concatenate([x[x.shape[0] - k:], x[: x.shape[0] - k]], axis=0)


def _fused_kernel(tgt_ref, x_ref, out_ref, oh_s, blank_s, y_s, bsh, ysh):
    pid = pl.program_id(0)
    nb_per_b = T // TBF
    b = pid // nb_per_b
    t0 = (pid % nb_per_b) * TBF

    @pl.when(pid % nb_per_b == 0)
    def _():
        tgt = tgt_ref[pl.ds(b, 1), :]
        iov = jax.lax.broadcasted_iota(jnp.int32, (U1, V), 1)
        oh_s[...] = jnp.where(iov == tgt.reshape(U1, 1), 1.0, 0.0)

    x = x_ref[...]
    m = jnp.max(x, axis=2, keepdims=True)
    s = jnp.sum(jnp.exp(x - m), axis=2, keepdims=True)
    lse = (m + jnp.log(s))[:, :, 0]
    blank = x[:, :, 0] - lse
    yv = jnp.sum(x * oh_s[...][None], axis=2) - lse

    blank_s[pl.ds(t0, TBF), pl.ds(b, 1), :] = blank[:, None, :]
    y_s[pl.ds(t0, TBF), pl.ds(b, 1), :] = yv[:, None, :]

    @pl.when(pid == NBLK - 1)
    def _():
        pad = jnp.full((DP - T, B, U1), NEG, jnp.float32)
        bp = jnp.concatenate([blank_s[...], pad], axis=0)
        yp = jnp.concatenate([y_s[...], pad], axis=0)
        iota_u = jax.lax.broadcasted_iota(jnp.int32, (DP, B, U1), 2)
        yp = jnp.where(iota_u == U, NEG, yp)
        for k in (1, 2, 4, 8, 16):
            mask = (iota_u & k) != 0
            bp = jnp.where(mask, _roll0(bp, k), bp)
            yp = jnp.where(mask, _roll0(yp, k), yp)
        bsh[...] = bp
        ysh[...] = yp

        iu3 = jax.lax.broadcasted_iota(jnp.int32, (1, B, U1), 2)
        a0 = jnp.where(iu3 == 0, 0.0, NEG)

        def body(d, a):
            bcol = bsh[pl.ds(d - 1, 1)]
            ycol = ysh[pl.ds(d - 1, 1)]
            c = a + ycol
            cs = jnp.concatenate(
                [jnp.full((1, B, 1), NEG, jnp.float32), c[:, :, :U]], axis=2)
            t1 = a + bcol
            mx = jnp.maximum(t1, cs)
            return mx + jnp.log1p(jnp.exp(-jnp.abs(t1 - cs)))

        a = jax.lax.fori_loop(1, D, body, a0)
        loglik = a[:, :, U1 - 1] + bsh[pl.ds(D - 1, 1)][:, :, U1 - 1]
        out_ref[...] = -jnp.sum(loglik, axis=1, keepdims=True) / B


def kernel(logits, targets, logit_lengths, target_lengths):
    x = logits.reshape(B * T, U1, V)
    tgt = jnp.concatenate(
        [targets.astype(jnp.int32), jnp.full((B, 1), -1, jnp.int32)], axis=1)
    out = pl.pallas_call(
        _fused_kernel,
        grid=(NBLK,),
        in_specs=[
            pl.BlockSpec((B, U1), lambda i: (0, 0)),
            pl.BlockSpec((TBF, U1, V), lambda i: (i, 0, 0)),
        ],
        out_specs=pl.BlockSpec((1, 1), lambda i: (0, 0)),
        out_shape=jax.ShapeDtypeStruct((1, 1), jnp.float32),
        scratch_shapes=[
            pltpu.VMEM((U1, V), jnp.float32),
            pltpu.VMEM((T, B, U1), jnp.float32),
            pltpu.VMEM((T, B, U1), jnp.float32),
            pltpu.VMEM((DP, B, U1), jnp.float32),
            pltpu.VMEM((DP, B, U1), jnp.float32),
        ],
    )(tgt, x)
    return out[0, 0]
